# baseline (speedup 1.0000x reference)
import functools

import jax
import jax.numpy as jnp
from jax import lax
from jax.experimental import pallas as pl
from jax.experimental.pallas import tpu as pltpu
from jax.experimental.pallas import tpu_sc as plsc

N_SRC = 5000
N_OUT = 10000
D = 512
NW = 32
REG = 312
REG_LAST = N_OUT - (NW - 1) * REG
CH = 104
TAIL = REG_LAST - 3 * CH
IDX_PAD = 8208
SENTINEL = 1 << 20
ZROW = N_SRC
SRC_LEN = 336


def _unpool_body(x_hbm, idx_hbm, out_hbm, idx_v, src_v, rows_v, rows_t, sem):
    cid = lax.axis_index("c")
    sid = lax.axis_index("s")
    wid = sid * 2 + cid
    r0 = pl.multiple_of(wid * REG, 8)
    is_last = wid == NW - 1
    reg = jnp.where(is_last, REG_LAST, REG)

    with jax.named_scope("stage_idx"):
        pltpu.sync_copy(idx_hbm, idx_v)

    zfill = jnp.full((16,), ZROW, jnp.int32)
    for j in range(SRC_LEN // 16):
        src_v[pl.ds(j * 16, 16)] = zfill

    def lower_bound(target):
        lo = jnp.int32(0)
        for s in (4096, 2048, 1024, 512, 256, 128, 64, 32, 16, 8, 4, 2, 1):
            v = idx_v[pl.ds(lo + s - 1, 16)][0]
            lo = jnp.where(v < target, lo + s, lo)
        return lo

    e_lo = lower_bound(r0)
    e_hi = lower_bound(r0 + reg)

    lanes = lax.iota(jnp.int32, 16)

    def scan_step(k, carry):
        off = k * 16
        a = idx_v[pl.ds(off, 16)]
        b = idx_v[pl.ds(off + 1, 16)]
        m = (a != b) & (a >= r0) & (a < r0 + reg)
        plsc.store_scatter(src_v, [a - r0], lanes + off, mask=m)
        return carry

    with jax.named_scope("scan"):
        lax.fori_loop(e_lo // 16, (e_hi + 15) // 16, scan_step, 0)

    with jax.named_scope("pull"):
        for c in range(3):
            pltpu.async_copy(
                x_hbm.at[src_v.at[pl.ds(c * CH, CH)]], rows_v, sem).wait()
            pltpu.sync_copy(rows_v, out_hbm.at[pl.ds(r0 + c * CH, CH)])

    @pl.when(is_last)
    def _tail():
        pltpu.async_copy(x_hbm.at[src_v.at[pl.ds(3 * CH, TAIL)]], rows_t, sem).wait()
        pltpu.sync_copy(rows_t, out_hbm.at[pl.ds(r0 + 3 * CH, TAIL)])


_unpool = functools.partial(
    pl.kernel,
    out_type=jax.ShapeDtypeStruct((N_OUT, D), jnp.float32),
    mesh=plsc.VectorSubcoreMesh(core_axis_name="c", subcore_axis_name="s"),
    compiler_params=pltpu.CompilerParams(
        needs_layout_passes=False, use_tc_tiling_on_sc=False),
    scratch_types=[
        pltpu.VMEM((IDX_PAD,), jnp.int32),
        pltpu.VMEM((SRC_LEN,), jnp.int32),
        pltpu.VMEM((CH, D), jnp.float32),
        pltpu.VMEM((TAIL, D), jnp.float32),
        pltpu.SemaphoreType.DMA,
    ],
)(_unpool_body)


def kernel(x, A, up_A, idx):
    x_pad = jnp.concatenate([x, jnp.zeros((8, D), x.dtype)], axis=0)
    idx_pad = jnp.concatenate([
        idx.astype(jnp.int32),
        jnp.full((IDX_PAD - N_SRC,), SENTINEL, jnp.int32),
    ])
    new_x = _unpool(x_pad, idx_pad)
    return (new_x, up_A)


# speedup vs baseline: 1.9458x; 1.9458x over previous
import functools

import jax
import jax.numpy as jnp
from jax import lax
from jax.experimental import pallas as pl
from jax.experimental.pallas import tpu as pltpu
from jax.experimental.pallas import tpu_sc as plsc

N_SRC = 5000
N_OUT = 10000
D = 512
NW = 32
REG = 312
REG_LAST = N_OUT - (NW - 1) * REG
CH = 104
TAIL = REG_LAST - 3 * CH
IDX_PAD = 8208
SENTINEL = 1 << 20
NPAD = 512
SRC_LEN = 336


def _unpool_body(x_hbm, idx_hbm, out_hbm, idx_v, src_v, rows_v, rows_t, sem):
    cid = lax.axis_index("c")
    sid = lax.axis_index("s")
    wid = sid * 2 + cid
    r0 = pl.multiple_of(wid * REG, 8)
    is_last = wid == NW - 1
    reg = jnp.where(is_last, REG_LAST, REG)

    with jax.named_scope("stage_idx"):
        pltpu.sync_copy(idx_hbm, idx_v)

    lanes0 = lax.iota(jnp.int32, 16)
    for j in range(SRC_LEN // 16):
        zfill = N_SRC + ((wid * 16 + j * 16 + lanes0) & (NPAD - 1))
        src_v[pl.ds(j * 16, 16)] = zfill

    def lower_bound(target):
        lo = jnp.int32(0)
        for s in (4096, 2048, 1024, 512, 256, 128, 64, 32, 16, 8, 4, 2, 1):
            v = idx_v[pl.ds(lo + s - 1, 16)][0]
            lo = jnp.where(v < target, lo + s, lo)
        return lo

    e_lo = lower_bound(r0)
    e_hi = lower_bound(r0 + reg)

    lanes = lax.iota(jnp.int32, 16)

    def scan_step(k, carry):
        off = k * 16
        a = idx_v[pl.ds(off, 16)]
        b = idx_v[pl.ds(off + 1, 16)]
        m = (a != b) & (a >= r0) & (a < r0 + reg)
        plsc.store_scatter(src_v, [a - r0], lanes + off, mask=m)
        return carry

    with jax.named_scope("scan"):
        lax.fori_loop(e_lo // 16, (e_hi + 15) // 16, scan_step, 0)

    with jax.named_scope("pull"):
        for c in range(3):
            pltpu.async_copy(
                x_hbm.at[src_v.at[pl.ds(c * CH, CH)]], rows_v, sem).wait()
            pltpu.sync_copy(rows_v, out_hbm.at[pl.ds(r0 + c * CH, CH)])

    @pl.when(is_last)
    def _tail():
        pltpu.async_copy(x_hbm.at[src_v.at[pl.ds(3 * CH, TAIL)]], rows_t, sem).wait()
        pltpu.sync_copy(rows_t, out_hbm.at[pl.ds(r0 + 3 * CH, TAIL)])


_unpool = functools.partial(
    pl.kernel,
    out_type=jax.ShapeDtypeStruct((N_OUT, D), jnp.float32),
    mesh=plsc.VectorSubcoreMesh(core_axis_name="c", subcore_axis_name="s"),
    compiler_params=pltpu.CompilerParams(needs_layout_passes=False),
    scratch_types=[
        pltpu.VMEM((IDX_PAD,), jnp.int32),
        pltpu.VMEM((SRC_LEN,), jnp.int32),
        pltpu.VMEM((CH, D), jnp.float32),
        pltpu.VMEM((TAIL, D), jnp.float32),
        pltpu.SemaphoreType.DMA,
    ],
)(_unpool_body)


def kernel(x, A, up_A, idx):
    x_pad = jnp.concatenate([x, jnp.zeros((NPAD, D), x.dtype)], axis=0)
    idx_pad = jnp.concatenate([
        idx.astype(jnp.int32),
        jnp.full((IDX_PAD - N_SRC,), SENTINEL, jnp.int32),
    ])
    new_x = _unpool(x_pad, idx_pad)
    return (new_x, up_A)


# speedup vs baseline: 1.9458x; 1.0000x over previous
import functools

import jax
import jax.numpy as jnp
from jax import lax
from jax.experimental import pallas as pl
from jax.experimental.pallas import tpu as pltpu
from jax.experimental.pallas import tpu_sc as plsc

N_SRC = 5000
N_OUT = 10000
D = 512
NW = 32
REG = 312
REG_LAST = N_OUT - (NW - 1) * REG
CH = 104
TAIL = REG_LAST - 3 * CH
IDX_PAD = 8208
SENTINEL = 1 << 20
NPAD = 512
SRC_LEN = 336


def _unpool_body(x_hbm, idx_hbm, out_hbm, idx_v, src_v, rows_v, rows_t, sem):
    cid = lax.axis_index("c")
    sid = lax.axis_index("s")
    wid = sid * 2 + cid
    r0 = pl.multiple_of(wid * REG, 8)
    is_last = wid == NW - 1
    reg = jnp.where(is_last, REG_LAST, REG)

    with jax.named_scope("stage_idx"):
        pltpu.sync_copy(idx_hbm, idx_v)

    lanes0 = lax.iota(jnp.int32, 16)
    for j in range(SRC_LEN // 16):
        zfill = N_SRC + ((wid * 16 + j * 16 + lanes0) & (NPAD - 1))
        src_v[pl.ds(j * 16, 16)] = zfill

    def lower_bound(target):
        lo = jnp.int32(0)
        for s in (4096, 2048, 1024, 512, 256, 128, 64, 32, 16, 8, 4, 2, 1):
            v = idx_v[pl.ds(lo + s - 1, 16)][0]
            lo = jnp.where(v < target, lo + s, lo)
        return lo

    e_lo = lower_bound(r0)
    e_hi = lower_bound(r0 + reg)

    lanes = lax.iota(jnp.int32, 16)

    def scan_step(k, carry):
        off = k * 16
        a = idx_v[pl.ds(off, 16)]
        b = idx_v[pl.ds(off + 1, 16)]
        m = (a != b) & (a >= r0) & (a < r0 + reg)
        plsc.store_scatter(src_v, [a - r0], lanes + off, mask=m)
        return carry

    with jax.named_scope("scan"):
        lax.fori_loop(e_lo // 16, (e_hi + 15) // 16, scan_step, 0)

    with jax.named_scope("pull"):
        for c in range(3):
            pltpu.async_copy(
                x_hbm.at[src_v.at[pl.ds(c * CH, CH)]], rows_v, sem).wait()
            pltpu.sync_copy(rows_v, out_hbm.at[pl.ds(r0 + c * CH, CH)])

    @pl.when(is_last)
    def _tail():
        pltpu.async_copy(x_hbm.at[src_v.at[pl.ds(3 * CH, TAIL)]], rows_t, sem).wait()
        pltpu.sync_copy(rows_t, out_hbm.at[pl.ds(r0 + 3 * CH, TAIL)])


_unpool = functools.partial(
    pl.kernel,
    out_type=jax.ShapeDtypeStruct((N_OUT, D), jnp.float32),
    mesh=plsc.VectorSubcoreMesh(core_axis_name="c", subcore_axis_name="s"),
    compiler_params=pltpu.CompilerParams(needs_layout_passes=False),
    cost_estimate=pl.CostEstimate(
        flops=0, bytes_accessed=45_000_000, transcendentals=0),
    scratch_types=[
        pltpu.VMEM((IDX_PAD,), jnp.int32),
        pltpu.VMEM((SRC_LEN,), jnp.int32),
        pltpu.VMEM((CH, D), jnp.float32),
        pltpu.VMEM((TAIL, D), jnp.float32),
        pltpu.SemaphoreType.DMA,
    ],
)(_unpool_body)


def kernel(x, A, up_A, idx):
    x_pad = jnp.concatenate([x, jnp.zeros((NPAD, D), x.dtype)], axis=0)
    idx_pad = jnp.concatenate([
        idx.astype(jnp.int32),
        jnp.full((IDX_PAD - N_SRC,), SENTINEL, jnp.int32),
    ])
    new_x = _unpool(x_pad, idx_pad)
    return (new_x, up_A)
